# baseline (device time: 11055 ns/iter reference)
import jax
import jax.numpy as jnp
from jax import lax
from jax.experimental import pallas as pl
from jax.experimental.pallas import tpu as pltpu

K = 8
NEG = float(jnp.finfo(jnp.float32).min)
ROWS = 256
RB = ROWS // 4

_OFFSETS = [
    (ox, oy, oz)
    for ox in (0, 1)
    for oy in (0, 1)
    for oz in (0, 1)
    if (ox, oy, oz) != (0, 0, 0)
]


def _topk_desc(v, k):
    cols = []
    for _ in range(k):
        mx = jnp.max(v, axis=1, keepdims=True)
        cols.append(mx)
        v = jnp.where(v == mx, NEG, v)
    return jnp.concatenate(cols, axis=1)


def kernel(x):
    m, n = x.shape

    def body(x_ref, out_ref, peers_ref, send_sems, recv_sems):
        my_x = lax.axis_index("x")
        my_y = lax.axis_index("y")
        my_z = lax.axis_index("z")

        def dest(ox, oy, oz):
            return ((my_x + ox) % 2, (my_y + oy) % 2, (my_z + oz) % 2)

        barrier_sem = pltpu.get_barrier_semaphore()
        for off in _OFFSETS:
            pl.semaphore_signal(
                barrier_sem,
                inc=1,
                device_id=dest(*off),
                device_id_type=pl.DeviceIdType.MESH,
            )
        pl.semaphore_wait(barrier_sem, len(_OFFSETS))

        my_b = 2 * my_x + my_y
        peers_ref[0, :, :] = _topk_desc(x_ref[pl.ds(my_b * RB, RB), :], K)

        rdmas = []
        for i, (ox, oy, oz) in enumerate(_OFFSETS):
            slot = 4 * ox + 2 * oy + oz
            rdma = pltpu.make_async_remote_copy(
                src_ref=peers_ref.at[0],
                dst_ref=peers_ref.at[slot],
                send_sem=send_sems.at[i],
                recv_sem=recv_sems.at[i],
                device_id=dest(ox, oy, oz),
                device_id_type=pl.DeviceIdType.MESH,
            )
            rdma.start()
            rdmas.append(rdma)
        for rdma in rdmas:
            rdma.wait_recv()

        both = jnp.concatenate(
            [
                jnp.concatenate(
                    [peers_ref[2 * p, :, :], peers_ref[2 * p + 1, :, :]], axis=1
                )
                for p in range(4)
            ],
            axis=0,
        )
        merged = _topk_desc(both, K)
        for p in range(4):
            ox, oy = p // 2, p % 2
            b = 2 * ((my_x + ox) % 2) + (my_y + oy) % 2
            out_ref[pl.ds(b * RB, RB), :] = merged[p * RB : (p + 1) * RB, :]

        for rdma in rdmas:
            rdma.wait_send()

    return pl.pallas_call(
        body,
        out_shape=jax.ShapeDtypeStruct((ROWS, K), jnp.float32),
        in_specs=[pl.BlockSpec(memory_space=pltpu.VMEM)],
        out_specs=pl.BlockSpec(memory_space=pltpu.VMEM),
        scratch_shapes=[
            pltpu.VMEM((8, RB, K), jnp.float32),
            pltpu.SemaphoreType.DMA((len(_OFFSETS),)),
            pltpu.SemaphoreType.DMA((len(_OFFSETS),)),
        ],
        compiler_params=pltpu.CompilerParams(collective_id=0),
    )(x)
